# baseline (speedup 1.0000x reference)
import functools

import jax
import jax.numpy as jnp
from jax import lax
from jax.experimental import pallas as pl
from jax.experimental.pallas import tpu as pltpu
from jax.experimental.pallas import tpu_sc as plsc

VOCAB = 1000000
EMBED_DIM = 64
BATCH = 4096
HIST = 200

NUM_CORES = 2
NUM_SUBCORES = 16
NUM_WORKERS = NUM_CORES * NUM_SUBCORES

TOTAL = BATCH * HIST
PER_WORKER = TOTAL // NUM_WORKERS
CHUNK = 1600
NUM_CHUNKS = PER_WORKER // CHUNK


def _embed_kernel(idx_hbm, table_hbm, out_hbm, idx_v, rows_v, gsem):
    wid = lax.axis_index("s") * NUM_CORES + lax.axis_index("c")
    wbase = wid * PER_WORKER

    def body(g, carry):
        base = wbase + g * CHUNK
        pltpu.sync_copy(idx_hbm.at[pl.ds(base, CHUNK)], idx_v)
        pltpu.async_copy(table_hbm.at[idx_v], rows_v, gsem).wait()
        pltpu.sync_copy(rows_v, out_hbm.at[pl.ds(base, CHUNK)])
        return carry

    lax.fori_loop(0, NUM_CHUNKS, body, 0)


@jax.jit
def _embed(idx_flat, table):
    mesh = plsc.VectorSubcoreMesh(
        core_axis_name="c", subcore_axis_name="s",
        num_cores=NUM_CORES, num_subcores=NUM_SUBCORES,
    )
    run = functools.partial(
        pl.kernel,
        out_type=jax.ShapeDtypeStruct((TOTAL, EMBED_DIM), jnp.float32),
        mesh=mesh,
        scratch_types=[
            pltpu.VMEM((CHUNK,), jnp.int32),
            pltpu.VMEM((CHUNK, EMBED_DIM), jnp.float32),
            pltpu.SemaphoreType.DMA,
        ],
        compiler_params=pltpu.CompilerParams(use_tc_tiling_on_sc=False),
    )(_embed_kernel)
    return run(idx_flat, table)


def kernel(x, table):
    out = _embed(x.reshape(-1), table)
    return out.reshape(BATCH, HIST, EMBED_DIM)


# speedup vs baseline: 1.0023x; 1.0023x over previous
import functools

import jax
import jax.numpy as jnp
from jax import lax
from jax.experimental import pallas as pl
from jax.experimental.pallas import tpu as pltpu
from jax.experimental.pallas import tpu_sc as plsc

VOCAB = 1000000
EMBED_DIM = 64
BATCH = 4096
HIST = 200

NUM_CORES = 2
NUM_SUBCORES = 16
NUM_WORKERS = NUM_CORES * NUM_SUBCORES

TOTAL = BATCH * HIST
PER_WORKER = TOTAL // NUM_WORKERS
CHUNK = 800
NUM_CHUNKS = PER_WORKER // CHUNK
NBUF = 2


def _embed_kernel(idx_hbm, table_hbm, out_hbm, idx_all, rows_v, gsems, ssems):
    wid = lax.axis_index("s") * NUM_CORES + lax.axis_index("c")
    wbase = wid * PER_WORKER

    pltpu.sync_copy(idx_hbm.at[pl.ds(wbase, PER_WORKER)], idx_all)

    def gather_copy(g, b):
        src = table_hbm.at[idx_all.at[pl.ds(g * CHUNK, CHUNK)]]
        return pltpu.make_async_copy(src, rows_v.at[b], gsems[b])

    def store_copy(g, b):
        dst = out_hbm.at[pl.ds(wbase + g * CHUNK, CHUNK)]
        return pltpu.make_async_copy(rows_v.at[b], dst, ssems[b])

    def visit(g, b):
        gather_copy(g, b).wait()
        store_copy(g, b).start()
        h = g + 1
        hb = (b + 1) % NBUF

        @pl.when(h < NUM_CHUNKS)
        def _():
            @pl.when(h >= NBUF)
            def _():
                store_copy(h - NBUF, hb).wait()
            gather_copy(h, hb).start()

    gather_copy(0, 0).start()

    def pair(p, carry):
        for b in range(NBUF):
            visit(NBUF * p + b, b)
        return carry

    lax.fori_loop(0, NUM_CHUNKS // NBUF, pair, 0)


@jax.jit
def _embed(idx_flat, table):
    mesh = plsc.VectorSubcoreMesh(
        core_axis_name="c", subcore_axis_name="s",
        num_cores=NUM_CORES, num_subcores=NUM_SUBCORES,
    )
    run = functools.partial(
        pl.kernel,
        out_type=jax.ShapeDtypeStruct((TOTAL, EMBED_DIM), jnp.float32),
        mesh=mesh,
        scratch_types=[
            pltpu.VMEM((PER_WORKER,), jnp.int32),
            pltpu.VMEM((NBUF, CHUNK, EMBED_DIM), jnp.float32),
            [pltpu.SemaphoreType.DMA] * NBUF,
            [pltpu.SemaphoreType.DMA] * NBUF,
        ],
        compiler_params=pltpu.CompilerParams(use_tc_tiling_on_sc=False),
    )(_embed_kernel)
    return run(idx_flat, table)


def kernel(x, table):
    out = _embed(x.reshape(-1), table)
    return out.reshape(BATCH, HIST, EMBED_DIM)


# speedup vs baseline: 1.0064x; 1.0041x over previous
import functools

import jax
import jax.numpy as jnp
from jax import lax
from jax.experimental import pallas as pl
from jax.experimental.pallas import tpu as pltpu
from jax.experimental.pallas import tpu_sc as plsc

VOCAB = 1000000
EMBED_DIM = 64
BATCH = 4096
HIST = 200

NUM_CORES = 2
NUM_SUBCORES = 16
NUM_WORKERS = NUM_CORES * NUM_SUBCORES

TOTAL = BATCH * HIST
PER_WORKER = TOTAL // NUM_WORKERS
CHUNK = 400
NUM_CHUNKS = PER_WORKER // CHUNK
NBUF = 4
DEPTH = NBUF - 1


def _embed_kernel(idx_hbm, table_hbm, out_hbm, idx_all, rows_v, gsems, ssems):
    wid = lax.axis_index("s") * NUM_CORES + lax.axis_index("c")
    wbase = wid * PER_WORKER

    pltpu.sync_copy(idx_hbm.at[pl.ds(wbase, PER_WORKER)], idx_all)

    def gather_copy(g, b):
        src = table_hbm.at[idx_all.at[pl.ds(g * CHUNK, CHUNK)]]
        return pltpu.make_async_copy(src, rows_v.at[b], gsems[b])

    def store_copy(g, b):
        dst = out_hbm.at[pl.ds(wbase + g * CHUNK, CHUNK)]
        return pltpu.make_async_copy(rows_v.at[b], dst, ssems[b])

    def visit(g, b):
        gather_copy(g, b).wait()
        store_copy(g, b).start()
        h = g + DEPTH
        hb = (b + DEPTH) % NBUF

        @pl.when(h < NUM_CHUNKS)
        def _():
            @pl.when(h >= NBUF)
            def _():
                store_copy(h - NBUF, hb).wait()
            gather_copy(h, hb).start()

    for h in range(DEPTH):
        gather_copy(h, h).start()

    def group(p, carry):
        for b in range(NBUF):
            visit(NBUF * p + b, b)
        return carry

    lax.fori_loop(0, NUM_CHUNKS // NBUF, group, 0)

    for b in range(NBUF):
        store_copy(NUM_CHUNKS - NBUF + b, b).wait()


@jax.jit
def _embed(idx_flat, table):
    mesh = plsc.VectorSubcoreMesh(
        core_axis_name="c", subcore_axis_name="s",
        num_cores=NUM_CORES, num_subcores=NUM_SUBCORES,
    )
    run = functools.partial(
        pl.kernel,
        out_type=jax.ShapeDtypeStruct((TOTAL, EMBED_DIM), jnp.float32),
        mesh=mesh,
        scratch_types=[
            pltpu.VMEM((PER_WORKER,), jnp.int32),
            pltpu.VMEM((NBUF, CHUNK, EMBED_DIM), jnp.float32),
            [pltpu.SemaphoreType.DMA] * NBUF,
            [pltpu.SemaphoreType.DMA] * NBUF,
        ],
        compiler_params=pltpu.CompilerParams(use_tc_tiling_on_sc=False),
    )(_embed_kernel)
    return run(idx_flat, table)


def kernel(x, table):
    out = _embed(x.reshape(-1), table)
    return out.reshape(BATCH, HIST, EMBED_DIM)
